# baseline (speedup 1.0000x reference)
import jax
import jax.numpy as jnp
from jax.experimental import pallas as pl
from jax.experimental.pallas import tpu as pltpu

_B, _T, _D, _H = 1024, 200, 64, 64
_S = 8


def _lstm_body(x_ref, wx_ref, wh_ref, b_ref, out_ref, h_ref, c_ref):
    t = pl.program_id(0)

    @pl.when(t == 0)
    def _init():
        h_ref[...] = jnp.zeros_like(h_ref)
        c_ref[...] = jnp.zeros_like(c_ref)

    wx = wx_ref[...]
    wh = wh_ref[...]
    b = b_ref[0:1, :]
    dn = (((1,), (0,)), ((), ()))

    xall = jnp.transpose(x_ref[...], (1, 0, 2))

    h = h_ref[...]
    c = c_ref[...]
    outs = []
    for s in range(_S):
        xg = jax.lax.dot_general(
            xall[s], wx, dn, preferred_element_type=jnp.float32)
        hg = jax.lax.dot_general(
            h, wh, dn, preferred_element_type=jnp.float32)
        gates = xg + hg + b
        ti = jnp.tanh(gates[:, 0 * _H:1 * _H])
        tf = jnp.tanh(gates[:, 1 * _H:2 * _H])
        tg = jnp.tanh(gates[:, 2 * _H:3 * _H])
        to = jnp.tanh(gates[:, 3 * _H:4 * _H])
        c = (ti * 0.5 + 0.5) * tg + (tf * 0.5 + 0.5) * c
        h = (to * 0.5 + 0.5) * jnp.tanh(c)
        outs.append(h)
    out_ref[...] = jnp.transpose(jnp.stack(outs, axis=0), (1, 0, 2))
    h_ref[...] = h
    c_ref[...] = c


def kernel(x, W_ih, W_hh, b_ih, b_hh):
    scale = jnp.concatenate([
        jnp.full((2 * _H,), 0.5, jnp.float32),
        jnp.ones((_H,), jnp.float32),
        jnp.full((_H,), 0.5, jnp.float32),
    ])
    wx = W_ih.T * scale[None, :]
    wh = W_hh.T * scale[None, :]
    b_row = jnp.broadcast_to(((b_ih + b_hh) * scale)[None, :], (8, 4 * _H))

    grid = (_T // _S,)

    out = pl.pallas_call(
        _lstm_body,
        grid=grid,
        in_specs=[
            pl.BlockSpec((_B, _S, _D), lambda t: (0, t, 0)),
            pl.BlockSpec((_D, 4 * _H), lambda t: (0, 0)),
            pl.BlockSpec((_H, 4 * _H), lambda t: (0, 0)),
            pl.BlockSpec((8, 4 * _H), lambda t: (0, 0)),
        ],
        out_specs=pl.BlockSpec((_B, _S, _H), lambda t: (0, t, 0)),
        out_shape=jax.ShapeDtypeStruct((_B, _T, _H), jnp.float32),
        scratch_shapes=[
            pltpu.VMEM((_B, _H), jnp.float32),
            pltpu.VMEM((_B, _H), jnp.float32),
        ],
        compiler_params=pltpu.CompilerParams(
            dimension_semantics=("arbitrary",),
        ),
    )(x, wx, wh, b_row)

    return out


# speedup vs baseline: 1.1797x; 1.1797x over previous
import jax
import jax.numpy as jnp
from jax.experimental import pallas as pl
from jax.experimental.pallas import tpu as pltpu

_B, _T, _D, _H = 1024, 200, 64, 64
_S = 8
_P = _B // 2


def _lstm_body(x_ref, wx_ref, wh_ref, b_ref, out_ref, h_ref, c_ref):
    t = pl.program_id(0)

    @pl.when(t == 0)
    def _init():
        h_ref[...] = jnp.zeros_like(h_ref)
        c_ref[...] = jnp.zeros_like(c_ref)

    wx = wx_ref[...]
    wh = wh_ref[...]
    b = b_ref[0:1, :]
    dn = (((1,), (0,)), ((), ()))
    w2 = 2 * _D

    h = h_ref[...]
    c = c_ref[...]
    for s in range(_S):
        xg = jax.lax.dot_general(
            x_ref[:, s * w2:(s + 1) * w2], wx, dn,
            preferred_element_type=jnp.float32)
        hg = jax.lax.dot_general(
            h.astype(jnp.bfloat16), wh, dn,
            preferred_element_type=jnp.float32)
        gates = xg + hg + b
        ti = jnp.tanh(gates[:, 0 * w2:1 * w2])
        tf = jnp.tanh(gates[:, 1 * w2:2 * w2])
        tg = jnp.tanh(gates[:, 2 * w2:3 * w2])
        to = jnp.tanh(gates[:, 3 * w2:4 * w2])
        c = (tf * 0.5 + 0.5) * c + (ti * 0.5 + 0.5) * tg
        h = (to * 0.5 + 0.5) * jnp.tanh(c)
        out_ref[:, s * w2:(s + 1) * w2] = h
    h_ref[...] = h
    c_ref[...] = c


def _block_diag(w):
    d = w.shape[0]
    w4 = w.reshape(d, 4, _H)
    out = jnp.zeros((2 * d, 4, 2, _H), dtype=w.dtype)
    out = out.at[:d, :, 0, :].set(w4)
    out = out.at[d:, :, 1, :].set(w4)
    return out.reshape(2 * d, 8 * _H)


def kernel(x, W_ih, W_hh, b_ih, b_hh):
    scale = jnp.concatenate([
        jnp.full((2 * _H,), 0.5, jnp.float32),
        jnp.ones((_H,), jnp.float32),
        jnp.full((_H,), 0.5, jnp.float32),
    ])
    wx_bd = _block_diag(W_ih.T * scale[None, :]).astype(jnp.bfloat16)
    wh_bd = _block_diag(W_hh.T * scale[None, :]).astype(jnp.bfloat16)
    b4 = ((b_ih + b_hh) * scale).reshape(4, _H)
    b_p = jnp.concatenate([b4, b4], axis=-1).reshape(8 * _H)
    b_row = jnp.broadcast_to(b_p[None, :], (8, 8 * _H))

    xp = jnp.concatenate([x[:_P], x[_P:]], axis=-1)
    xp = xp.reshape(_P, _T * 2 * _D).astype(jnp.bfloat16)

    grid = (_T // _S,)

    out = pl.pallas_call(
        _lstm_body,
        grid=grid,
        in_specs=[
            pl.BlockSpec((_P, _S * 2 * _D), lambda t: (0, t)),
            pl.BlockSpec((2 * _D, 8 * _H), lambda t: (0, 0)),
            pl.BlockSpec((2 * _H, 8 * _H), lambda t: (0, 0)),
            pl.BlockSpec((8, 8 * _H), lambda t: (0, 0)),
        ],
        out_specs=pl.BlockSpec((_P, _S * 2 * _H), lambda t: (0, t)),
        out_shape=jax.ShapeDtypeStruct((_P, _T * 2 * _H), jnp.float32),
        scratch_shapes=[
            pltpu.VMEM((_P, 2 * _H), jnp.float32),
            pltpu.VMEM((_P, 2 * _H), jnp.float32),
        ],
        compiler_params=pltpu.CompilerParams(
            dimension_semantics=("arbitrary",),
        ),
    )(xp, wx_bd, wh_bd, b_row)

    o3 = out.reshape(_P, _T, 2 * _H)
    return jnp.concatenate([o3[:, :, :_H], o3[:, :, _H:]], axis=0)


# speedup vs baseline: 1.8279x; 1.5494x over previous
import jax
import jax.numpy as jnp
from jax.experimental import pallas as pl
from jax.experimental.pallas import tpu as pltpu

_B, _T, _D, _H = 1024, 200, 64, 64
_S = 8
_P = _B // 2


def _lstm_body(x_ref, wx_ref, wh_ref, b_ref, out_ref, h_ref, c_ref):
    t = pl.program_id(0)

    @pl.when(t == 0)
    def _init():
        h_ref[...] = jnp.zeros_like(h_ref)
        c_ref[...] = jnp.zeros_like(c_ref)

    wx = wx_ref[...]
    wh = wh_ref[...]
    b = b_ref[0:1, :]
    dn = (((1,), (0,)), ((), ()))
    w2 = 2 * _H

    h = h_ref[...]
    c = c_ref[...]
    for s in range(_S):
        xa = x_ref[0:_P, s * _D:(s + 1) * _D]
        xb = x_ref[_P:_B, s * _D:(s + 1) * _D]
        xg = jax.lax.dot_general(
            jnp.concatenate([xa, xb], axis=-1), wx, dn,
            preferred_element_type=jnp.float32)
        hg = jax.lax.dot_general(
            h.astype(jnp.bfloat16), wh, dn,
            preferred_element_type=jnp.float32)
        gates = xg + hg + b
        ti = jnp.tanh(gates[:, 0 * w2:1 * w2])
        tf = jnp.tanh(gates[:, 1 * w2:2 * w2])
        tg = jnp.tanh(gates[:, 2 * w2:3 * w2])
        to = jnp.tanh(gates[:, 3 * w2:4 * w2])
        c = (tf * 0.5 + 0.5) * c + (ti * 0.5 + 0.5) * tg
        h = (to * 0.5 + 0.5) * jnp.tanh(c)
        out_ref[0:_P, s * _H:(s + 1) * _H] = h[:, :_H]
        out_ref[_P:_B, s * _H:(s + 1) * _H] = h[:, _H:]
    h_ref[...] = h
    c_ref[...] = c


def _block_diag(w):
    d = w.shape[0]
    w4 = w.reshape(d, 4, _H)
    out = jnp.zeros((2 * d, 4, 2, _H), dtype=w.dtype)
    out = out.at[:d, :, 0, :].set(w4)
    out = out.at[d:, :, 1, :].set(w4)
    return out.reshape(2 * d, 8 * _H)


def kernel(x, W_ih, W_hh, b_ih, b_hh):
    scale = jnp.concatenate([
        jnp.full((2 * _H,), 0.5, jnp.float32),
        jnp.ones((_H,), jnp.float32),
        jnp.full((_H,), 0.5, jnp.float32),
    ])
    wx_bd = _block_diag(W_ih.T * scale[None, :]).astype(jnp.bfloat16)
    wh_bd = _block_diag(W_hh.T * scale[None, :]).astype(jnp.bfloat16)
    b4 = ((b_ih + b_hh) * scale).reshape(4, _H)
    b_p = jnp.concatenate([b4, b4], axis=-1).reshape(8 * _H)
    b_row = jnp.broadcast_to(b_p[None, :], (8, 8 * _H))

    x2 = x.reshape(_B, _T * _D).astype(jnp.bfloat16)

    grid = (_T // _S,)

    out = pl.pallas_call(
        _lstm_body,
        grid=grid,
        in_specs=[
            pl.BlockSpec((_B, _S * _D), lambda t: (0, t)),
            pl.BlockSpec((2 * _D, 8 * _H), lambda t: (0, 0)),
            pl.BlockSpec((2 * _H, 8 * _H), lambda t: (0, 0)),
            pl.BlockSpec((8, 8 * _H), lambda t: (0, 0)),
        ],
        out_specs=pl.BlockSpec((_B, _S * _H), lambda t: (0, t)),
        out_shape=jax.ShapeDtypeStruct((_B, _T * _H), jnp.float32),
        scratch_shapes=[
            pltpu.VMEM((_P, 2 * _H), jnp.float32),
            pltpu.VMEM((_P, 2 * _H), jnp.float32),
        ],
        compiler_params=pltpu.CompilerParams(
            dimension_semantics=("arbitrary",),
        ),
    )(x2, wx_bd, wh_bd, b_row)

    return out.reshape(_B, _T, _H)


# speedup vs baseline: 2.2133x; 1.2108x over previous
import jax
import jax.numpy as jnp
from jax.experimental import pallas as pl
from jax.experimental.pallas import tpu as pltpu

_B, _T, _D, _H = 1024, 200, 64, 64
_S = 8
_P = _B // 2


def _lstm_body(x_ref, w_ref, b_ref, out_ref, h_ref, c_ref):
    t = pl.program_id(0)

    @pl.when(t == 0)
    def _init():
        h_ref[...] = jnp.zeros_like(h_ref)
        c_ref[...] = jnp.zeros_like(c_ref)

    wxh = w_ref[...]
    b = b_ref[0:1, :]
    dn = (((1,), (0,)), ((), ()))
    w2 = 2 * _H

    h = h_ref[...]
    c = c_ref[...]
    for s in range(_S):
        xa = x_ref[0:_P, s * _D:(s + 1) * _D]
        xb = x_ref[_P:_B, s * _D:(s + 1) * _D]
        lhs = jnp.concatenate([xa, xb, h.astype(jnp.bfloat16)], axis=-1)
        gates = jax.lax.dot_general(
            lhs, wxh, dn, preferred_element_type=jnp.float32) + b
        ti = jnp.tanh(gates[:, 0 * w2:1 * w2])
        tf = jnp.tanh(gates[:, 1 * w2:2 * w2])
        tg = jnp.tanh(gates[:, 2 * w2:3 * w2])
        to = jnp.tanh(gates[:, 3 * w2:4 * w2])
        c = (tf * 0.5 + 0.5) * c + (ti * 0.5 + 0.5) * tg
        h = (to * 0.5 + 0.5) * jnp.tanh(c)
        hb = h.astype(jnp.bfloat16)
        out_ref[0:_P, s * _H:(s + 1) * _H] = hb[:, :_H]
        out_ref[_P:_B, s * _H:(s + 1) * _H] = hb[:, _H:]
    h_ref[...] = h
    c_ref[...] = c


def _block_diag(w):
    d = w.shape[0]
    w4 = w.reshape(d, 4, _H)
    out = jnp.zeros((2 * d, 4, 2, _H), dtype=w.dtype)
    out = out.at[:d, :, 0, :].set(w4)
    out = out.at[d:, :, 1, :].set(w4)
    return out.reshape(2 * d, 8 * _H)


def kernel(x, W_ih, W_hh, b_ih, b_hh):
    scale = jnp.concatenate([
        jnp.full((2 * _H,), 0.5, jnp.float32),
        jnp.ones((_H,), jnp.float32),
        jnp.full((_H,), 0.5, jnp.float32),
    ])
    wx_bd = _block_diag(W_ih.T * scale[None, :]).astype(jnp.bfloat16)
    wh_bd = _block_diag(W_hh.T * scale[None, :]).astype(jnp.bfloat16)
    wxh = jnp.concatenate([wx_bd, wh_bd], axis=0)
    b4 = ((b_ih + b_hh) * scale).reshape(4, _H)
    b_p = jnp.concatenate([b4, b4], axis=-1).reshape(8 * _H)
    b_row = jnp.broadcast_to(b_p[None, :], (8, 8 * _H))

    x2 = x.reshape(_B, _T * _D).astype(jnp.bfloat16)

    grid = (_T // _S,)

    out = pl.pallas_call(
        _lstm_body,
        grid=grid,
        in_specs=[
            pl.BlockSpec((_B, _S * _D), lambda t: (0, t)),
            pl.BlockSpec((2 * _D + 2 * _H, 8 * _H), lambda t: (0, 0)),
            pl.BlockSpec((8, 8 * _H), lambda t: (0, 0)),
        ],
        out_specs=pl.BlockSpec((_B, _S * _H), lambda t: (0, t)),
        out_shape=jax.ShapeDtypeStruct((_B, _T * _H), jnp.bfloat16),
        scratch_shapes=[
            pltpu.VMEM((_P, 2 * _H), jnp.float32),
            pltpu.VMEM((_P, 2 * _H), jnp.float32),
        ],
        compiler_params=pltpu.CompilerParams(
            dimension_semantics=("arbitrary",),
        ),
    )(x2, wxh, b_row)

    return out.reshape(_B, _T, _H).astype(jnp.float32)
